# Initial kernel scaffold; baseline (speedup 1.0000x reference)
import jax
import jax.numpy as jnp
from jax.experimental import pallas as pl


def kernel(weights, indices):
    raise NotImplementedError("write your pallas kernel here")


# speedup vs baseline: 1.0948x; 1.0948x over previous
import functools

import jax
import jax.numpy as jnp
from jax import lax
from jax.experimental import pallas as pl
from jax.experimental.pallas import tpu as pltpu
from jax.experimental.pallas import tpu_sc as plsc

_B = 16384
_CHUNK = 128
_NC, _NS = 2, 16
_NW = _NC * _NS
_ROWS = _B // _CHUNK
_RPW = _ROWS // _NW


def _sc_gather(weights, idx2d):
    mesh = plsc.VectorSubcoreMesh(core_axis_name="c", subcore_axis_name="s")

    @functools.partial(
        pl.kernel,
        out_type=jax.ShapeDtypeStruct((_ROWS, _CHUNK), jnp.float32),
        mesh=mesh,
        scratch_types=[
            pltpu.VMEM((_RPW, _CHUNK), jnp.int32),
            pltpu.VMEM((_RPW, _CHUNK), jnp.float32),
            pltpu.SemaphoreType.DMA,
        ],
    )
    def gather_kernel(w_hbm, idx_hbm, out_hbm, idx_v, val_v, sem):
        wid = lax.axis_index("s") * _NC + lax.axis_index("c")
        base = wid * _RPW
        pltpu.sync_copy(idx_hbm.at[pl.ds(base, _RPW)], idx_v)
        copies = [
            pltpu.async_copy(w_hbm.at[idx_v.at[j]], val_v.at[j], sem)
            for j in range(_RPW)
        ]
        for c in copies:
            c.wait()
        pltpu.sync_copy(val_v, out_hbm.at[pl.ds(base, _RPW)])

    return gather_kernel(weights, idx2d)


def kernel(weights, indices):
    idx2d = indices.astype(jnp.int32).reshape(_ROWS, _CHUNK)
    return _sc_gather(weights, idx2d).reshape(_B)


# speedup vs baseline: 1.0953x; 1.0004x over previous
import functools

import jax
import jax.numpy as jnp
from jax import lax
from jax.experimental import pallas as pl
from jax.experimental.pallas import tpu as pltpu
from jax.experimental.pallas import tpu_sc as plsc

_B = 16384
_CHUNK = 128
_NC, _NS = 2, 16
_NW = _NC * _NS
_BPW = _B // _NW
_NCH = _BPW // _CHUNK


def _sc_gather(weights, indices):
    mesh = plsc.VectorSubcoreMesh(core_axis_name="c", subcore_axis_name="s")

    @functools.partial(
        pl.kernel,
        out_type=jax.ShapeDtypeStruct((_B,), jnp.float32),
        mesh=mesh,
        scratch_types=[
            pltpu.VMEM((_BPW,), jnp.int32),
            pltpu.VMEM((_BPW,), jnp.float32),
            pltpu.SemaphoreType.DMA,
        ],
    )
    def gather_kernel(w_hbm, idx_hbm, out_hbm, idx_v, val_v, sem):
        wid = lax.axis_index("s") * _NC + lax.axis_index("c")
        base = wid * _BPW
        pltpu.sync_copy(idx_hbm.at[pl.ds(base, _BPW)], idx_v)
        copies = [
            pltpu.async_copy(
                w_hbm.at[idx_v.at[pl.ds(j * _CHUNK, _CHUNK)]],
                val_v.at[pl.ds(j * _CHUNK, _CHUNK)],
                sem,
            )
            for j in range(_NCH)
        ]
        for c in copies:
            c.wait()
        pltpu.sync_copy(val_v, out_hbm.at[pl.ds(base, _BPW)])

    return gather_kernel(weights, indices)


def kernel(weights, indices):
    return _sc_gather(weights, indices.astype(jnp.int32))


# speedup vs baseline: 1.1056x; 1.0094x over previous
import functools

import jax
import jax.numpy as jnp
from jax import lax
from jax.experimental import pallas as pl
from jax.experimental.pallas import tpu as pltpu
from jax.experimental.pallas import tpu_sc as plsc

_B = 16384
_CHUNK = 128
_NC, _NS = 2, 16
_NW = _NC * _NS
_BPW = _B // _NW
_NCH = _BPW // _CHUNK


def _sc_gather(weights, indices):
    mesh = plsc.VectorSubcoreMesh(core_axis_name="c", subcore_axis_name="s")

    @functools.partial(
        pl.kernel,
        out_type=jax.ShapeDtypeStruct((_B,), jnp.float32),
        mesh=mesh,
        scratch_types=[
            pltpu.VMEM((_BPW,), jnp.int32),
            pltpu.VMEM((_BPW,), jnp.float32),
            [pltpu.SemaphoreType.DMA] * _NCH,
            pltpu.SemaphoreType.DMA,
        ],
    )
    def gather_kernel(w_hbm, idx_hbm, out_hbm, idx_v, val_v, gsems, osem):
        wid = lax.axis_index("s") * _NC + lax.axis_index("c")
        base = wid * _BPW
        pltpu.sync_copy(idx_hbm.at[pl.ds(base, _BPW)], idx_v)
        gathers = [
            pltpu.async_copy(
                w_hbm.at[idx_v.at[pl.ds(j * _CHUNK, _CHUNK)]],
                val_v.at[pl.ds(j * _CHUNK, _CHUNK)],
                gsems[j],
            )
            for j in range(_NCH)
        ]
        outs = []
        for j in range(_NCH):
            gathers[j].wait()
            outs.append(
                pltpu.async_copy(
                    val_v.at[pl.ds(j * _CHUNK, _CHUNK)],
                    out_hbm.at[pl.ds(base + j * _CHUNK, _CHUNK)],
                    osem,
                )
            )
        for c in outs:
            c.wait()

    return gather_kernel(weights, indices)


def kernel(weights, indices):
    return _sc_gather(weights, indices.astype(jnp.int32))
